# baseline (speedup 1.0000x reference)
import jax
import jax.numpy as jnp
from jax.experimental import pallas as pl


def _extract_body(x_ref, o_ref):
    b = pl.program_id(0)
    o_ref[pl.ds(b, 1), :] = x_ref[0:1, :]


def kernel(inputs):
    B, S, D = inputs.shape
    x2 = inputs.reshape(B * S, D)
    return pl.pallas_call(
        _extract_body,
        grid=(B,),
        in_specs=[pl.BlockSpec((8, D), lambda i: (i * (S // 8), 0))],
        out_specs=pl.BlockSpec((B, D), lambda i: (0, 0)),
        out_shape=jax.ShapeDtypeStruct((B, D), inputs.dtype),
    )(x2)


# speedup vs baseline: 1.6853x; 1.6853x over previous
import jax
import jax.numpy as jnp
from jax.experimental import pallas as pl
from jax.experimental.pallas import tpu as pltpu


def _extract_body(x_hbm_ref, o_ref, sem):
    B = o_ref.shape[0]
    copies = [
        pltpu.make_async_copy(x_hbm_ref.at[b, 0, :], o_ref.at[b], sem)
        for b in range(B)
    ]
    for c in copies:
        c.start()
    for c in copies:
        c.wait()


def kernel(inputs):
    B, S, D = inputs.shape
    return pl.pallas_call(
        _extract_body,
        in_specs=[pl.BlockSpec(memory_space=pl.ANY)],
        out_specs=pl.BlockSpec((B, D), lambda: (0, 0)),
        out_shape=jax.ShapeDtypeStruct((B, D), inputs.dtype),
        scratch_shapes=[pltpu.SemaphoreType.DMA],
    )(inputs)
